# baseline (speedup 1.0000x reference)
import functools

import jax
import jax.numpy as jnp
from jax import lax
from jax.experimental import pallas as pl
from jax.experimental.pallas import tpu as pltpu
from jax.experimental.pallas import tpu_sc as plsc

N_USERS = 1000000
N_ITEMS = 100000
K = 32
B = 16384
MU = 3.5

_INFO = plsc.get_sparse_core_info()
NC, NS, L = _INFO.num_cores, _INFO.num_subcores, _INFO.num_lanes
NW = NC * NS
BPW = B // NW
CB = 128
NCHUNK = BPW // CB
CGROUPS = CB // L


def _lfm_kernel(uidx_hbm, iidx_hbm, p_hbm, q_hbm, bu_hbm, bi_hbm, out_hbm,
                uidx_v, iidx_v, sup_v, p_v, q_v, bu_v, bi_v, o_v, sem):
    wid = lax.axis_index("s") * NC + lax.axis_index("c")
    base = wid * BPW

    pltpu.sync_copy(uidx_hbm.at[pl.ds(base, BPW)], uidx_v)
    pltpu.sync_copy(iidx_hbm.at[pl.ds(base, BPW)], iidx_v)

    lane = lax.iota(jnp.int32, L)

    def chunk(c, carry):
        c0 = c * CB
        def mkidx(j, carry):
            u16 = uidx_v[pl.ds(c0 + j * L, L)]
            i16 = iidx_v[pl.ds(c0 + j * L, L)]
            sup_v[0, pl.ds(j * L, L)] = u16 >> 2
            sup_v[1, pl.ds(j * L, L)] = i16 >> 2
            sup_v[2, pl.ds(j * L, L)] = u16 >> 7
            sup_v[3, pl.ds(j * L, L)] = i16 >> 7
            return carry

        lax.fori_loop(0, CB // L, mkidx, 0)

        cps = [
            pltpu.async_copy(p_hbm.at[sup_v.at[0]], p_v, sem),
            pltpu.async_copy(q_hbm.at[sup_v.at[1]], q_v, sem),
            pltpu.async_copy(bu_hbm.at[sup_v.at[2]], bu_v, sem),
            pltpu.async_copy(bi_hbm.at[sup_v.at[3]], bi_v, sem),
        ]
        for cp in cps:
            cp.wait()

        def group(g, carry):
            rows = g * L + lane
            u16 = uidx_v[pl.ds(c0 + g * L, L)]
            i16 = iidx_v[pl.ds(c0 + g * L, L)]
            acc = (MU + plsc.load_gather(bu_v, [rows, u16 & 127])
                   + plsc.load_gather(bi_v, [rows, i16 & 127]))
            usub = (u16 & 3) * K
            isub = (i16 & 3) * K
            for k in range(K):
                pk = plsc.load_gather(p_v, [rows, usub + k])
                qk = plsc.load_gather(q_v, [rows, isub + k])
                acc = acc + pk * qk
            o_v[pl.ds(c0 + g * L, L)] = acc
            return carry

        lax.fori_loop(0, CGROUPS, group, 0)
        return carry

    lax.fori_loop(0, NCHUNK, chunk, 0)
    pltpu.sync_copy(o_v, out_hbm.at[pl.ds(base, BPW)])


@jax.jit
def kernel(user_idx, item_idx, P, Q, b_u, b_i):
    mesh = plsc.VectorSubcoreMesh(core_axis_name="c", subcore_axis_name="s")
    run = functools.partial(
        pl.kernel,
        mesh=mesh,
        out_type=jax.ShapeDtypeStruct((B,), jnp.float32),
        scratch_types=[
            pltpu.VMEM((BPW,), jnp.int32),
            pltpu.VMEM((BPW,), jnp.int32),
            pltpu.VMEM((4, CB), jnp.int32),
            pltpu.VMEM((CB, 128), jnp.float32),
            pltpu.VMEM((CB, 128), jnp.float32),
            pltpu.VMEM((CB, 128), jnp.float32),
            pltpu.VMEM((CB, 128), jnp.float32),
            pltpu.VMEM((BPW,), jnp.float32),
            pltpu.SemaphoreType.DMA,
        ],
        compiler_params=pltpu.CompilerParams(
            needs_layout_passes=False, use_tc_tiling_on_sc=True),
    )(_lfm_kernel)
    bu_p = jnp.pad(b_u.reshape(-1), (0, 64)).reshape(-1, 128)
    bi_p = jnp.pad(b_i.reshape(-1), (0, 96)).reshape(-1, 128)
    return run(user_idx, item_idx, P.reshape(-1, 128), Q.reshape(-1, 128),
               bu_p, bi_p)
